# Initial kernel scaffold; baseline (speedup 1.0000x reference)
import functools

import jax
import jax.numpy as jnp
from jax import lax
from jax.experimental import pallas as pl
from jax.experimental.pallas import tpu as pltpu
from jax.experimental.pallas import tpu_sc as plsc

_NC, _NS = 2, 16
_NW = _NC * _NS

_BATCH = 4096
_HIST = 200
_D = 32
_B = _BATCH * _HIST
_B_PER_W = _B // _NW
_CHUNK = 1024
_NCHUNK = _B_PER_W // _CHUNK

_mesh = plsc.VectorSubcoreMesh(core_axis_name="c", subcore_axis_name="s")


@functools.partial(
    pl.kernel,
    out_type=jax.ShapeDtypeStruct((_B, _D), jnp.float32),
    mesh=_mesh,
    scratch_types=[
        pltpu.VMEM((_CHUNK,), jnp.int32),
        pltpu.VMEM((_CHUNK, _D), jnp.float32),
        pltpu.SemaphoreType.DMA,
    ],
)
def _gather_rows(idx_hbm, table_hbm, out_hbm, idx_v, rows_v, sem):
    wid = lax.axis_index("s") * _NC + lax.axis_index("c")
    base_w = wid * _B_PER_W

    def body(i, carry):
        base = base_w + i * _CHUNK
        pltpu.sync_copy(idx_hbm.at[pl.ds(base, _CHUNK)], idx_v)
        pltpu.async_copy(table_hbm.at[idx_v], rows_v, sem).wait()
        pltpu.sync_copy(rows_v, out_hbm.at[pl.ds(base, _CHUNK)])
        return carry

    lax.fori_loop(0, _NCHUNK, body, 0)


def kernel(X, position_table):
    idx = X.reshape(-1).astype(jnp.int32)
    out = _gather_rows(idx, position_table)
    return out.reshape(_BATCH, _HIST, _D)


# speedup vs baseline: 1.4591x; 1.4591x over previous
import functools

import jax
import jax.numpy as jnp
from jax import lax
from jax.experimental import pallas as pl
from jax.experimental.pallas import tpu as pltpu
from jax.experimental.pallas import tpu_sc as plsc

_NC, _NS = 2, 16
_NW = _NC * _NS

_BATCH = 4096
_HIST = 200
_D = 32
_B = _BATCH * _HIST
_B_PER_W = _B // _NW
_CHUNK = 1024
_NCHUNK = _B_PER_W // _CHUNK

_mesh = plsc.VectorSubcoreMesh(core_axis_name="c", subcore_axis_name="s")


@functools.partial(
    pl.kernel,
    out_type=jax.ShapeDtypeStruct((_B, _D), jnp.float32),
    mesh=_mesh,
    scratch_types=[
        pltpu.VMEM((_CHUNK,), jnp.int32),
        pltpu.VMEM((_CHUNK, _D), jnp.float32),
        pltpu.SemaphoreType.DMA,
    ],
    compiler_params=pltpu.CompilerParams(use_tc_tiling_on_sc=False),
)
def _gather_rows(idx_hbm, table_hbm, out_hbm, idx_v, rows_v, sem):
    wid = lax.axis_index("s") * _NC + lax.axis_index("c")
    base_w = wid * _B_PER_W

    def body(i, carry):
        base = base_w + i * _CHUNK
        pltpu.sync_copy(idx_hbm.at[pl.ds(base, _CHUNK)], idx_v)
        pltpu.async_copy(table_hbm.at[idx_v], rows_v, sem).wait()
        pltpu.sync_copy(rows_v, out_hbm.at[pl.ds(base, _CHUNK)])
        return carry

    lax.fori_loop(0, _NCHUNK, body, 0)


def kernel(X, position_table):
    idx = X.reshape(-1).astype(jnp.int32)
    out = _gather_rows(idx, position_table)
    return out.reshape(_BATCH, _HIST, _D)


# speedup vs baseline: 1.5014x; 1.0290x over previous
import functools

import jax
import jax.numpy as jnp
from jax import lax
from jax.experimental import pallas as pl
from jax.experimental.pallas import tpu as pltpu
from jax.experimental.pallas import tpu_sc as plsc

_NC, _NS = 2, 16
_NW = _NC * _NS

_BATCH = 4096
_HIST = 200
_D = 32
_B = _BATCH * _HIST
_B_PER_W = _B // _NW
_CHUNK = 640
_NCHUNK = _B_PER_W // _CHUNK
_NBUF = 4
_K = 2

assert (_NCHUNK - _NBUF) % _NBUF == 0

_mesh = plsc.VectorSubcoreMesh(core_axis_name="c", subcore_axis_name="s")


@functools.partial(
    pl.kernel,
    out_type=jax.ShapeDtypeStruct((_B, _D), jnp.float32),
    mesh=_mesh,
    scratch_types=(
        [pltpu.VMEM((_B_PER_W,), jnp.int32),
         pltpu.VMEM((_NBUF, _CHUNK, _D), jnp.float32)]
        + [pltpu.SemaphoreType.DMA] * (2 * _NBUF)
    ),
    compiler_params=pltpu.CompilerParams(use_tc_tiling_on_sc=False),
)
def _gather_rows(idx_hbm, table_hbm, out_hbm, idx_all, rows, *sems):
    gsem = sems[:_NBUF]
    ssem = sems[_NBUF:]
    wid = lax.axis_index("s") * _NC + lax.axis_index("c")
    base_w = wid * _B_PER_W

    pltpu.sync_copy(idx_hbm.at[pl.ds(base_w, _B_PER_W)], idx_all)

    def gather_desc(j, b):
        idx_slice = idx_all.at[pl.ds(j * _CHUNK, _CHUNK)]
        return pltpu.make_async_copy(table_hbm.at[idx_slice], rows.at[b],
                                     gsem[b])

    def store_desc(j, b):
        dst = out_hbm.at[pl.ds(base_w + j * _CHUNK, _CHUNK)]
        return pltpu.make_async_copy(rows.at[b], dst, ssem[b])

    for j in range(_K):
        gather_desc(j, j % _NBUF).start()

    for i in range(_NBUF - _K):
        j = i + _K
        gather_desc(j, j % _NBUF).start()
        b = i % _NBUF
        gather_desc(i, b).wait()
        store_desc(i, b).start()

    def gen_body(g, carry):
        i0 = (_NBUF - _K) + g * _NBUF
        for t in range(_NBUF):
            i = i0 + t
            j = i + _K
            bj = (_NBUF - _K + t + _K) % _NBUF
            store_desc(j - _NBUF, bj).wait()
            gather_desc(j, bj).start()
            b = (_NBUF - _K + t) % _NBUF
            gather_desc(i, b).wait()
            store_desc(i, b).start()
        return carry

    ngen = (_NCHUNK - _NBUF) // _NBUF
    lax.fori_loop(0, ngen, gen_body, 0)

    for i in range(_NCHUNK - _K, _NCHUNK):
        b = i % _NBUF
        gather_desc(i, b).wait()
        store_desc(i, b).start()

    for i in range(_NCHUNK - _NBUF, _NCHUNK):
        store_desc(i, i % _NBUF).wait()


def kernel(X, position_table):
    idx = X.reshape(-1).astype(jnp.int32)
    out = _gather_rows(idx, position_table)
    return out.reshape(_BATCH, _HIST, _D)


# speedup vs baseline: 1.5778x; 1.0509x over previous
import functools

import jax
import jax.numpy as jnp
from jax import lax
from jax.experimental import pallas as pl
from jax.experimental.pallas import tpu as pltpu
from jax.experimental.pallas import tpu_sc as plsc

_NC, _NS = 2, 16
_NW = _NC * _NS

_BATCH = 4096
_HIST = 200
_D = 32
_BB = _BATCH // _NW
_HC = 8
_NCHUNK = _HIST // _HC
_NBUF = 3
_K = 2

_mesh = plsc.VectorSubcoreMesh(core_axis_name="c", subcore_axis_name="s")


@functools.partial(
    pl.kernel,
    out_type=jax.ShapeDtypeStruct((_HIST, _BATCH, _D), jnp.float32),
    mesh=_mesh,
    scratch_types=(
        [pltpu.VMEM((_HIST, _BB), jnp.int32),
         pltpu.VMEM((_NBUF, _HC, _BB, _D), jnp.float32)]
        + [pltpu.SemaphoreType.DMA] * (2 * _NBUF)
    ),
    compiler_params=pltpu.CompilerParams(use_tc_tiling_on_sc=False),
)
def _gather_rows(xt_hbm, table_hbm, out_hbm, idx_all, rows, *sems):
    gsem = sems[:_NBUF]
    ssem = sems[_NBUF:]
    wid = lax.axis_index("s") * _NC + lax.axis_index("c")
    b0 = wid * _BB

    pltpu.sync_copy(xt_hbm.at[:, pl.ds(b0, _BB)], idx_all)

    def gather_start(c, b):
        for t in range(_HC):
            pltpu.make_async_copy(
                table_hbm.at[idx_all.at[c * _HC + t]],
                rows.at[b].at[t],
                gsem[b],
            ).start()

    def gather_wait(c, b):
        for t in range(_HC):
            pltpu.make_async_copy(
                table_hbm.at[idx_all.at[c * _HC + t]],
                rows.at[b].at[t],
                gsem[b],
            ).wait()

    def store_start(c, b):
        for t in range(_HC):
            pltpu.make_async_copy(
                rows.at[b].at[t],
                out_hbm.at[c * _HC + t].at[pl.ds(b0, _BB)],
                ssem[b],
            ).start()

    def store_wait(c, b):
        for t in range(_HC):
            pltpu.make_async_copy(
                rows.at[b].at[t],
                out_hbm.at[c * _HC + t].at[pl.ds(b0, _BB)],
                ssem[b],
            ).wait()

    for c in range(_K):
        gather_start(c, c % _NBUF)

    for c in range(_NBUF - _K):
        gather_start(c + _K, (c + _K) % _NBUF)
        gather_wait(c, c % _NBUF)
        store_start(c, c % _NBUF)

    def gen_body(g, carry):
        c0 = (_NBUF - _K) + g * _NBUF
        for t in range(_NBUF):
            c = c0 + t
            j = c + _K
            bj = (t + _NBUF) % _NBUF
            store_wait(j - _NBUF, bj)
            gather_start(j, bj)
            b = (_NBUF - _K + t) % _NBUF
            gather_wait(c, b)
            store_start(c, b)
        return carry

    ngen = (_NCHUNK - _NBUF) // _NBUF
    lax.fori_loop(0, ngen, gen_body, 0)

    done = (_NBUF - _K) + ngen * _NBUF
    for c in range(done, _NCHUNK - _K):
        j = c + _K
        store_wait(j - _NBUF, j % _NBUF)
        gather_start(j, j % _NBUF)
        gather_wait(c, c % _NBUF)
        store_start(c, c % _NBUF)

    for c in range(_NCHUNK - _K, _NCHUNK):
        gather_wait(c, c % _NBUF)
        store_start(c, c % _NBUF)

    for c in range(_NCHUNK - _NBUF, _NCHUNK):
        store_wait(c, c % _NBUF)


def kernel(X, position_table):
    xt = X.T.astype(jnp.int32)
    out_t = _gather_rows(xt, position_table)
    return out_t.transpose(1, 0, 2)
